# baseline (speedup 1.0000x reference)
import functools

import jax
import jax.numpy as jnp
from jax import lax
from jax.experimental import pallas as pl
from jax.experimental.pallas import tpu as pltpu
from jax.experimental.pallas import tpu_sc as plsc

_S = 2048
_H = 16
_PAD = 4096
_BLK = 8
_W = 2064


def _expand_bias(rev_flat):
    mesh = plsc.VectorSubcoreMesh(core_axis_name="c", subcore_axis_name="s")
    blocks_per_worker = (_S // _BLK) // 2
    npairs = blocks_per_worker // 2

    @functools.partial(
        pl.kernel,
        mesh=mesh,
        out_type=jax.ShapeDtypeStruct((1, _H, _S, _S), jnp.float32),
        scratch_types=[
            pltpu.VMEM((4 * _W,), jnp.float32),
            pltpu.VMEM((4, _BLK, _S), jnp.float32),
            pltpu.SemaphoreType.DMA,
            pltpu.SemaphoreType.DMA,
            pltpu.SemaphoreType.DMA,
            pltpu.SemaphoreType.DMA,
            pltpu.SemaphoreType.DMA,
            pltpu.SemaphoreType.DMA,
            pltpu.SemaphoreType.DMA,
            pltpu.SemaphoreType.DMA,
        ],
    )
    def body(tab_hbm, out_hbm, wins, stages,
             wsem0, wsem1, wsem2, wsem3, osem0, osem1, osem2, osem3):
        wsems = (wsem0, wsem1, wsem2, wsem3)
        osems = (osem0, osem1, osem2, osem3)
        cid = lax.axis_index("c")
        sid = lax.axis_index("s")
        wid = sid * 2 + cid
        h = wid // 2
        t0 = (wid % 2) * blocks_per_worker

        def win_copy(s, ti, lead):
            start = pl.multiple_of(
                h * _PAD + (_S - 1) - ti * _BLK - lead, 16)
            return pltpu.make_async_copy(
                tab_hbm.at[pl.ds(start, _W)],
                wins.at[pl.ds(s * _W, _W)], wsems[s])

        def out_copy(s, ti):
            return pltpu.make_async_copy(
                stages.at[s],
                out_hbm.at[0, h, pl.ds(ti * _BLK, _BLK), :],
                osems[s],
            )

        def build(s, lead):
            for k in range(_S // 128):
                for r in range(_BLK):
                    base = lead - r + k * 128
                    for c in range(8):
                        stages[s, r, pl.ds(k * 128 + c * 16, 16)] = (
                            wins[pl.ds(s * _W + base + c * 16, 16)]
                        )

        leads = (15, 7, 15, 7)
        for s in range(4):
            win_copy(s, t0 + s, leads[s]).start()

        nquads = blocks_per_worker // 4

        def loop(g, carry):
            ti = t0 + 4 * g
            for s in range(4):
                win_copy(s, ti + s, leads[s]).wait()

                @pl.when(g > 0)
                def _():
                    out_copy(s, ti + s - 4).wait()

                build(s, leads[s])
                out_copy(s, ti + s).start()

                @pl.when(g < nquads - 1)
                def _():
                    win_copy(s, ti + s + 4, leads[s]).start()

            return carry

        lax.fori_loop(0, nquads, loop, 0)
        for s in range(4):
            out_copy(s, t0 + blocks_per_worker - 4 + s).wait()

    return body(rev_flat)


def kernel(seq_len, bias_table):
    del seq_len
    rev = bias_table[::-1, :]
    rev = jnp.concatenate(
        [rev, jnp.zeros((_PAD - rev.shape[0], _H), bias_table.dtype)], axis=0)
    rev_flat = jnp.transpose(rev, (1, 0)).reshape(_H * _PAD)
    return _expand_bias(rev_flat)


# speedup vs baseline: 1.1009x; 1.1009x over previous
import functools

import jax
import jax.numpy as jnp
from jax import lax
from jax.experimental import pallas as pl
from jax.experimental.pallas import tpu as pltpu
from jax.experimental.pallas import tpu_sc as plsc

_S = 2048
_H = 16
_PAD = 4096
_BLK = 8
_W = 2064


def _expand_bias(rev_flat):
    mesh = plsc.VectorSubcoreMesh(core_axis_name="c", subcore_axis_name="s")
    blocks_per_worker = (_S // _BLK) // 2
    npairs = blocks_per_worker // 2

    @functools.partial(
        pl.kernel,
        mesh=mesh,
        out_type=jax.ShapeDtypeStruct((1, _H, _S, _S), jnp.float32),
        scratch_types=[
            pltpu.VMEM((_W,), jnp.float32),
            pltpu.VMEM((_W,), jnp.float32),
            pltpu.VMEM((_BLK, _S), jnp.float32),
            pltpu.VMEM((_BLK, _S), jnp.float32),
            pltpu.SemaphoreType.DMA,
            pltpu.SemaphoreType.DMA,
            pltpu.SemaphoreType.DMA,
            pltpu.SemaphoreType.DMA,
        ],
    )
    def body(tab_hbm, out_hbm, win0, win1, stage0, stage1,
             wsem0, wsem1, osem0, osem1):
        cid = lax.axis_index("c")
        sid = lax.axis_index("s")
        wid = sid * 2 + cid
        h = wid // 2
        t0 = (wid % 2) * blocks_per_worker

        def win_copy(win, wsem, ti, lead):
            start = pl.multiple_of(
                h * _PAD + (_S - 1) - ti * _BLK - lead, 16)
            return pltpu.make_async_copy(
                tab_hbm.at[pl.ds(start, _W)], win, wsem)

        def out_copy(stage, osem, ti):
            return pltpu.make_async_copy(
                stage,
                out_hbm.at[0, h, pl.ds(ti * _BLK, _BLK), :],
                osem,
            )

        def build(stage, win, lead):
            for k in range(_S // 128):
                for r in range(_BLK):
                    base = lead - r + k * 128
                    for c in range(8):
                        stage[r, pl.ds(k * 128 + c * 16, 16)] = (
                            win[pl.ds(base + c * 16, 16)]
                        )

        win_copy(win0, wsem0, t0, 15).start()
        win_copy(win1, wsem1, t0 + 1, 7).start()

        def loop(g, carry):
            ti = t0 + 2 * g
            win_copy(win0, wsem0, ti, 15).wait()

            @pl.when(g > 0)
            def _():
                out_copy(stage0, osem0, ti - 2).wait()

            build(stage0, win0, 15)
            out_copy(stage0, osem0, ti).start()

            @pl.when(g < npairs - 1)
            def _():
                win_copy(win0, wsem0, ti + 2, 15).start()

            win_copy(win1, wsem1, ti + 1, 7).wait()

            @pl.when(g > 0)
            def _():
                out_copy(stage1, osem1, ti - 1).wait()

            build(stage1, win1, 7)
            out_copy(stage1, osem1, ti + 1).start()

            @pl.when(g < npairs - 1)
            def _():
                win_copy(win1, wsem1, ti + 3, 7).start()

            return carry

        lax.fori_loop(0, npairs, loop, 0)
        out_copy(stage0, osem0, t0 + blocks_per_worker - 2).wait()
        out_copy(stage1, osem1, t0 + blocks_per_worker - 1).wait()

    return body(rev_flat)


def kernel(seq_len, bias_table):
    del seq_len
    n = bias_table.shape[0]
    d = jnp.arange(_PAD)[None, :]
    h = jnp.arange(_H)[:, None]
    idx = jnp.where(d < n, (n - 1 - d) * _H + h, -1)
    rev_flat = jnp.take(
        bias_table.reshape(-1), idx.reshape(-1), mode="fill", fill_value=0.0)
    return _expand_bias(rev_flat)


# speedup vs baseline: 1.1327x; 1.0289x over previous
import functools

import jax
import jax.numpy as jnp
from jax import lax
from jax.experimental import pallas as pl
from jax.experimental.pallas import tpu as pltpu
from jax.experimental.pallas import tpu_sc as plsc

_S = 2048
_H = 16
_PAD = 4096
_BLK = 8
_W = 2064


def _expand_bias(rev_flat):
    mesh = plsc.VectorSubcoreMesh(core_axis_name="c", subcore_axis_name="s")
    blocks_per_worker = (_S // _BLK) // 2
    npairs = blocks_per_worker // 2

    @functools.partial(
        pl.kernel,
        mesh=mesh,
        out_type=jax.ShapeDtypeStruct((1, _H, _S, _S), jnp.float32),
        scratch_types=[
            pltpu.VMEM((_W,), jnp.float32),
            pltpu.VMEM((_W,), jnp.float32),
            pltpu.VMEM((_BLK, _S), jnp.float32),
            pltpu.VMEM((_BLK, _S), jnp.float32),
            pltpu.SemaphoreType.DMA,
            pltpu.SemaphoreType.DMA,
            pltpu.SemaphoreType.DMA,
            pltpu.SemaphoreType.DMA,
        ],
    )
    def body(tab_hbm, out_hbm, win0, win1, stage0, stage1,
             wsem0, wsem1, osem0, osem1):
        cid = lax.axis_index("c")
        sid = lax.axis_index("s")
        wid = sid * 2 + cid
        h = wid // 2
        t0 = (wid % 2) * blocks_per_worker

        def win_copy(win, wsem, ti, lead):
            start = pl.multiple_of(
                h * _PAD + (_S - 1) - ti * _BLK - lead, 16)
            return pltpu.make_async_copy(
                tab_hbm.at[pl.ds(start, _W)], win, wsem)

        def out_copy(stage, osem, ti):
            return pltpu.make_async_copy(
                stage,
                out_hbm.at[0, h, pl.ds(ti * _BLK, _BLK), :],
                osem,
            )

        def build(stage, win, lead):
            for k in range(_S // 128):
                for r in range(_BLK):
                    base = lead - r + k * 128
                    for c in range(8):
                        stage[r, pl.ds(k * 128 + c * 16, 16)] = (
                            win[pl.ds(base + c * 16, 16)]
                        )

        win_copy(win0, wsem0, t0, 15).start()
        win_copy(win1, wsem1, t0 + 1, 7).start()

        def loop(g, carry):
            ti = t0 + 2 * g
            win_copy(win0, wsem0, ti, 15).wait()

            @pl.when(g > 0)
            def _():
                out_copy(stage0, osem0, ti - 2).wait()

            build(stage0, win0, 15)
            out_copy(stage0, osem0, ti).start()

            @pl.when(g < npairs - 1)
            def _():
                win_copy(win0, wsem0, ti + 2, 15).start()

            win_copy(win1, wsem1, ti + 1, 7).wait()

            @pl.when(g > 0)
            def _():
                out_copy(stage1, osem1, ti - 1).wait()

            build(stage1, win1, 7)
            out_copy(stage1, osem1, ti + 1).start()

            @pl.when(g < npairs - 1)
            def _():
                win_copy(win1, wsem1, ti + 3, 7).start()

            return carry

        lax.fori_loop(0, npairs, loop, 0)
        out_copy(stage0, osem0, t0 + blocks_per_worker - 2).wait()
        out_copy(stage1, osem1, t0 + blocks_per_worker - 1).wait()

    return body(rev_flat)


def kernel(seq_len, bias_table):
    del seq_len
    rev = bias_table[::-1, :]
    rev = jnp.concatenate(
        [rev, jnp.zeros((_PAD - rev.shape[0], _H), bias_table.dtype)], axis=0)
    rev_flat = jnp.transpose(rev, (1, 0)).reshape(_H * _PAD)
    return _expand_bias(rev_flat)
